# baseline (speedup 1.0000x reference)
import jax
import jax.numpy as jnp
from jax.experimental import pallas as pl
from jax.experimental.pallas import tpu as pltpu

_DB = 384
_BB = 2


def _add_kernel(x_ref, row_ref, o_ref):
    o_ref[:, :, :] = x_ref[:, :, :] + row_ref[:, :][None]


def kernel(x, row_embed):
    B, L, D = x.shape
    table = row_embed[:L]
    grid = (D // _DB, B // _BB)
    return pl.pallas_call(
        _add_kernel,
        grid=grid,
        in_specs=[
            pl.BlockSpec((_BB, L, _DB), lambda d, b: (b, 0, d)),
            pl.BlockSpec((L, _DB), lambda d, b: (0, d)),
        ],
        out_specs=pl.BlockSpec((_BB, L, _DB), lambda d, b: (b, 0, d)),
        out_shape=jax.ShapeDtypeStruct((B, L, D), x.dtype),
        compiler_params=pltpu.CompilerParams(
            dimension_semantics=("arbitrary", "arbitrary"),
        ),
    )(x, table)


# speedup vs baseline: 1.0405x; 1.0405x over previous
import jax
import jax.numpy as jnp
from jax.experimental import pallas as pl
from jax.experimental.pallas import tpu as pltpu


def _add_kernel(x_ref, row_ref, o_ref):
    o_ref[:, :] = x_ref[:, :] + row_ref[:, :]


def kernel(x, row_embed):
    B, L, D = x.shape
    table = row_embed[:L]
    x2 = x.reshape(B * L, D)
    out = pl.pallas_call(
        _add_kernel,
        grid=(B,),
        in_specs=[
            pl.BlockSpec((L, D), lambda b: (b, 0)),
            pl.BlockSpec((L, D), lambda b: (0, 0)),
        ],
        out_specs=pl.BlockSpec((L, D), lambda b: (b, 0)),
        out_shape=jax.ShapeDtypeStruct((B * L, D), x.dtype),
        compiler_params=pltpu.CompilerParams(
            dimension_semantics=("arbitrary",),
        ),
    )(x2, table)
    return out.reshape(B, L, D)


# speedup vs baseline: 1.0434x; 1.0027x over previous
import jax
import jax.numpy as jnp
from jax.experimental import pallas as pl
from jax.experimental.pallas import tpu as pltpu


def _add_kernel(x_ref, row_ref, o_ref):
    o_ref[:, :] = x_ref[:, :] + row_ref[:, :]


def kernel(x, row_embed):
    B, L, D = x.shape
    table = row_embed[:L]
    x2 = x.reshape(B * L, D)
    out = pl.pallas_call(
        _add_kernel,
        grid=(B,),
        in_specs=[
            pl.BlockSpec((L, D), lambda b: (b, 0)),
            pl.BlockSpec((L, D), lambda b: (0, 0)),
        ],
        out_specs=pl.BlockSpec((L, D), lambda b: (b, 0)),
        out_shape=jax.ShapeDtypeStruct((B * L, D), x.dtype),
        compiler_params=pltpu.CompilerParams(
            dimension_semantics=("parallel",),
        ),
    )(x2, table)
    return out.reshape(B, L, D)
